# baseline (speedup 1.0000x reference)
import functools

import jax
import jax.numpy as jnp
from jax import lax
from jax.experimental import pallas as pl
from jax.experimental.pallas import tpu as pltpu

_G = 64
_NBUF = 8
_D = 128


def _body(x_hbm, p_ref, o_hbm, xbuf, obuf, insem, outsem, *, nstep):
    def in_copy(step, slot):
        return pltpu.make_async_copy(
            x_hbm.at[pl.ds(step * _G, _G)], xbuf.at[slot], insem.at[slot])

    for s in range(_NBUF):
        in_copy(s, s).start()

    def outer(i, _):
        for b in range(_NBUF):
            step = i * _NBUF + b
            in_copy(step, b).wait()
            y = lax.dot_general(
                p_ref[...], xbuf[b],
                dimension_numbers=(((2,), (2,)), ((0,), (0,))),
                preferred_element_type=jnp.float32,
            )
            obuf[pl.ds(step * _G, _G)] = y.reshape(_G, _D)

            @pl.when(step + _NBUF < nstep)
            def _():
                in_copy(step + _NBUF, b).start()
        return 0

    lax.fori_loop(0, nstep // _NBUF, outer, 0)
    final = pltpu.make_async_copy(obuf, o_hbm, outsem)
    final.start()
    final.wait()


def kernel(x, projection):
    b, n, d = x.shape
    rows = b * n
    groups = rows // d
    nstep = groups // _G
    xf = x.reshape(groups, d, d)
    pb = jnp.broadcast_to(projection.reshape(1, 1, d), (_G, 1, d))
    out = pl.pallas_call(
        functools.partial(_body, nstep=nstep),
        in_specs=[
            pl.BlockSpec(memory_space=pl.ANY),
            pl.BlockSpec(memory_space=pltpu.VMEM),
        ],
        out_specs=pl.BlockSpec(memory_space=pl.ANY),
        out_shape=jax.ShapeDtypeStruct((groups, d), jnp.float32),
        scratch_shapes=[
            pltpu.VMEM((_NBUF, _G, d, d), jnp.float32),
            pltpu.VMEM((groups, d), jnp.float32),
            pltpu.SemaphoreType.DMA((_NBUF,)),
            pltpu.SemaphoreType.DMA,
        ],
    )(xf, pb)
    return out.reshape(b, n, 1)
